# baseline (speedup 1.0000x reference)
import functools

import jax
import jax.numpy as jnp
from jax import lax
from jax.experimental import pallas as pl
from jax.experimental.pallas import tpu as pltpu
from jax.experimental.pallas import tpu_sc as plsc

L = 16
D = 4096
VPR = D // L
NROWS = 32 * 16 * 16
NC, NS = 2, 16
NW = NC * NS
ROWS_PER_W = NROWS // NW
RB = 8
NT = ROWS_PER_W // RB
BISECT = 14


def _recip(x):
    xi = lax.bitcast_convert_type(x, jnp.int32)
    r = lax.bitcast_convert_type(jnp.int32(0x7EF311C3) - xi, jnp.float32)
    for _ in range(3):
        r = r * (2.0 - x * r)
    return r


UMAX = 8
UCP = 4
UOUT = 8


def _row_sparsemax(zbuf, obuf, cbuf, r):
    def mx_body(i, accs):
        base = i * (L * UMAX)
        return tuple(
            jnp.maximum(a, zbuf[r, pl.ds(base + j * L, L)])
            for j, a in enumerate(accs))
    accs = lax.fori_loop(
        0, VPR // UMAX, mx_body,
        (jnp.full((L,), -jnp.inf, jnp.float32),) * UMAX)
    accm = functools.reduce(jnp.maximum, accs)
    mx = jnp.max(accm)
    lo0 = mx - 1.0

    def cp_body(i, cnt_vec):
        base = i * (L * UCP)
        for j in range(UCP):
            v = zbuf[r, pl.ds(base + j * L, L)]
            m = v > lo0
            csum = plsc.cumsum(jnp.where(m, 1, 0))
            plsc.store_scatter(cbuf, (cnt_vec + csum - 1,), v, mask=m)
            cnt_vec = cnt_vec + plsc.all_reduce_population_count(m)
        return cnt_vec
    cnt_vec = lax.fori_loop(
        0, VPR // UCP, cp_body, jnp.zeros((L,), jnp.int32))
    cnt = cnt_vec[0]
    cbuf[pl.ds(cnt, L)] = jnp.full((L,), lo0 - 1.0, jnp.float32)
    nv = lax.shift_right_logical(cnt + (L - 1), 4)

    def bis_body(_, lohi):
        lo, hi = lohi
        t = 0.5 * (lo + hi)
        def f_body(i, acc):
            v = cbuf[pl.ds(i * L, L)]
            return acc + jnp.maximum(v - t, 0.0)
        acc = lax.fori_loop(0, nv, f_body, jnp.zeros((L,), jnp.float32))
        ge = (jnp.sum(acc) - 1.0) >= 0.0
        return jnp.where(ge, t, lo), jnp.where(ge, hi, t)
    lo, _hi = lax.fori_loop(0, BISECT, bis_body, (lo0, mx))

    def nf_body(i, acc):
        v = cbuf[pl.ds(i * L, L)]
        m = v > lo
        sacc, kacc = acc
        return (sacc + jnp.where(m, v, 0.0), kacc + jnp.where(m, 1.0, 0.0))
    sacc, kacc = lax.fori_loop(
        0, nv, nf_body,
        (jnp.zeros((L,), jnp.float32), jnp.zeros((L,), jnp.float32)))
    tau = (jnp.sum(sacc) - 1.0) * _recip(jnp.maximum(jnp.sum(kacc), 1.0))

    def out_body(i, carry):
        base = i * (L * UOUT)
        for j in range(UOUT):
            v = zbuf[r, pl.ds(base + j * L, L)]
            obuf[r, pl.ds(base + j * L, L)] = jnp.maximum(v - tau, 0.0)
        return carry
    lax.fori_loop(0, VPR // UOUT, out_body, 0)


def _sc_body(x_hbm, out_hbm, zbuf, obuf, cbuf):
    wid = lax.axis_index("s") * NC + lax.axis_index("c")
    row0 = wid * ROWS_PER_W

    def block_body(tidx, carry):
        base = row0 + tidx * RB
        pltpu.sync_copy(x_hbm.at[pl.ds(base, RB)], zbuf)
        def row_body(r, c):
            _row_sparsemax(zbuf, obuf, cbuf, r)
            return c
        lax.fori_loop(0, RB, row_body, 0)
        pltpu.sync_copy(obuf, out_hbm.at[pl.ds(base, RB)])
        return carry
    lax.fori_loop(0, NT, block_body, 0)


@jax.jit
def kernel(input):
    x = input.reshape(NROWS, D)
    mesh = plsc.VectorSubcoreMesh(
        core_axis_name="c", subcore_axis_name="s", num_cores=NC,
        num_subcores=NS)
    run = functools.partial(
        pl.kernel,
        out_type=jax.ShapeDtypeStruct((NROWS, D), jnp.float32),
        mesh=mesh,
        compiler_params=pltpu.CompilerParams(needs_layout_passes=False),
        scratch_types=[
            pltpu.VMEM((RB, D), jnp.float32),
            pltpu.VMEM((RB, D), jnp.float32),
            pltpu.VMEM((D + L,), jnp.float32),
        ],
    )(_sc_body)
    return run(x).reshape(input.shape)


# speedup vs baseline: 2.2093x; 2.2093x over previous
import functools

import jax
import jax.numpy as jnp
from jax import lax
from jax.experimental import pallas as pl
from jax.experimental.pallas import tpu as pltpu
from jax.experimental.pallas import tpu_sc as plsc

L = 16
D = 4096
VPR = D // L
NROWS = 32 * 16 * 16
NC, NS = 2, 16
NW = NC * NS
ROWS_PER_W = NROWS // NW
RB = 4
NT = ROWS_PER_W // RB
BISECT = 12

UMAX = 8
UCP = 8
UOUT = 8


def _splat_max(x):
    return plsc.cummax(lax.rev(plsc.cummax(x), (0,)))


def _vsum_nonneg(x):
    return plsc.cummax(lax.rev(plsc.cumsum(x), (0,)))


def _vrecip(x):
    xi = plsc.bitcast(x, jnp.int32)
    r = plsc.bitcast(jnp.full((L,), 0x7EF311C3, jnp.int32) - xi, jnp.float32)
    for _ in range(3):
        r = r * (2.0 - x * r)
    return r


def _row_sparsemax(zbuf, obuf, cbuf, r):
    def mx_body(i, accs):
        base = i * (L * UMAX)
        return tuple(
            jnp.maximum(a, zbuf[r, pl.ds(base + j * L, L)])
            for j, a in enumerate(accs))
    accs = lax.fori_loop(
        0, VPR // UMAX, mx_body,
        (jnp.full((L,), -jnp.inf, jnp.float32),) * UMAX)
    accm = functools.reduce(jnp.maximum, accs)
    mxs = _splat_max(accm)
    lo0 = mxs - 1.0

    def cp_body(i, cnt_vec):
        base = i * (L * UCP)
        vs, ms, csums, pcs = [], [], [], []
        for j in range(UCP):
            v = zbuf[r, pl.ds(base + j * L, L)]
            m = v > lo0
            vs.append(v)
            ms.append(m)
            csums.append(plsc.cumsum(jnp.where(m, 1, 0)))
            pcs.append(plsc.all_reduce_population_count(m))
        off = cnt_vec
        for j in range(UCP):
            plsc.store_scatter(cbuf, (off + csums[j] - 1,), vs[j], mask=ms[j])
            off = off + pcs[j]
        return off
    cnt_vec = lax.fori_loop(
        0, VPR // UCP, cp_body, jnp.zeros((L,), jnp.int32))
    cnt = cnt_vec[0]
    cbuf[pl.ds(cnt, L)] = lo0 - 1.0
    nv = lax.shift_right_logical(cnt + (L - 1), 4)

    def bis_body(_, lohi):
        lo, hi = lohi
        t = 0.5 * (lo + hi)
        def f_body(i, acc):
            v = cbuf[pl.ds(i * L, L)]
            return acc + jnp.maximum(v - t, 0.0)
        acc = lax.fori_loop(0, nv, f_body, jnp.zeros((L,), jnp.float32))
        ge = _vsum_nonneg(acc) >= 1.0
        return jnp.where(ge, t, lo), jnp.where(ge, hi, t)
    lo, _hi = lax.fori_loop(0, BISECT, bis_body, (lo0, mxs))

    def nf_body(i, acc):
        v = cbuf[pl.ds(i * L, L)]
        m = v > lo
        sacc, kacc = acc
        return (sacc + jnp.where(m, v - lo0, 0.0),
                kacc + jnp.where(m, 1.0, 0.0))
    sacc, kacc = lax.fori_loop(
        0, nv, nf_body,
        (jnp.zeros((L,), jnp.float32), jnp.zeros((L,), jnp.float32)))
    ks = jnp.maximum(_vsum_nonneg(kacc), 1.0)
    tau = (_vsum_nonneg(sacc) + ks * lo0 - 1.0) * _vrecip(ks)

    def out_body(i, carry):
        base = i * (L * UOUT)
        for j in range(UOUT):
            v = zbuf[r, pl.ds(base + j * L, L)]
            obuf[r, pl.ds(base + j * L, L)] = jnp.maximum(v - tau, 0.0)
        return carry
    lax.fori_loop(0, VPR // UOUT, out_body, 0)


def _sc_body(x_hbm, out_hbm, zb0, zb1, ob0, ob1, cbuf, si0, si1, so0, so1):
    wid = lax.axis_index("s") * NC + lax.axis_index("c")
    row0 = wid * ROWS_PER_W
    zb, ob, si, so = (zb0, zb1), (ob0, ob1), (si0, si1), (so0, so1)

    for b in range(2):
        pltpu.async_copy(x_hbm.at[pl.ds(row0 + b * RB, RB)], zb[b], si[b])

    def outer(o, carry):
        for b in range(2):
            t = 2 * o + b
            base = row0 + t * RB
            pltpu.make_async_copy(x_hbm.at[pl.ds(base, RB)], zb[b],
                                  si[b]).wait()

            @pl.when(o > 0)
            def _():
                pltpu.make_async_copy(
                    ob[b], out_hbm.at[pl.ds(base - 2 * RB, RB)],
                    so[b]).wait()

            def row_body(r, c):
                _row_sparsemax(zb[b], ob[b], cbuf, r)
                return c
            lax.fori_loop(0, RB, row_body, 0)
            pltpu.async_copy(ob[b], out_hbm.at[pl.ds(base, RB)], so[b])

            @pl.when(t + 2 < NT)
            def _():
                pltpu.async_copy(x_hbm.at[pl.ds(base + 2 * RB, RB)], zb[b],
                                 si[b])
        return carry
    lax.fori_loop(0, NT // 2, outer, 0)

    for b in range(2):
        base = row0 + (NT - 2 + b) * RB
        pltpu.make_async_copy(ob[b], out_hbm.at[pl.ds(base, RB)],
                              so[b]).wait()


@jax.jit
def kernel(input):
    x = input.reshape(NROWS, D)
    mesh = plsc.VectorSubcoreMesh(
        core_axis_name="c", subcore_axis_name="s", num_cores=NC,
        num_subcores=NS)
    run = functools.partial(
        pl.kernel,
        out_type=jax.ShapeDtypeStruct((NROWS, D), jnp.float32),
        mesh=mesh,
        compiler_params=pltpu.CompilerParams(needs_layout_passes=False),
        scratch_types=[
            pltpu.VMEM((RB, D), jnp.float32),
            pltpu.VMEM((RB, D), jnp.float32),
            pltpu.VMEM((RB, D), jnp.float32),
            pltpu.VMEM((RB, D), jnp.float32),
            pltpu.VMEM((D + L,), jnp.float32),
            pltpu.SemaphoreType.DMA,
            pltpu.SemaphoreType.DMA,
            pltpu.SemaphoreType.DMA,
            pltpu.SemaphoreType.DMA,
        ],
    )(_sc_body)
    return run(x).reshape(input.shape)


# speedup vs baseline: 3.1516x; 1.4265x over previous
import functools

import jax
import jax.numpy as jnp
from jax import lax
from jax.experimental import pallas as pl
from jax.experimental.pallas import tpu as pltpu
from jax.experimental.pallas import tpu_sc as plsc

L = 16
D = 4096
VPR = D // L
NROWS = 32 * 16 * 16
NC, NS = 2, 16
NW = NC * NS
SC_ROWS = 4096
TC_ROWS = NROWS - SC_ROWS
ROWS_PER_W = SC_ROWS // NW
RB = 4
NT = ROWS_PER_W // RB
TCB = 128
BISECT = 12

UMAX = 8
UCP = 8
UOUT = 8


def _splat_max(x):
    return plsc.cummax(lax.rev(plsc.cummax(x), (0,)))


def _vsum_nonneg(x):
    return plsc.cummax(lax.rev(plsc.cumsum(x), (0,)))


def _vrecip(x):
    xi = plsc.bitcast(x, jnp.int32)
    r = plsc.bitcast(jnp.full((L,), 0x7EF311C3, jnp.int32) - xi, jnp.float32)
    for _ in range(3):
        r = r * (2.0 - x * r)
    return r


def _row_sparsemax(zbuf, obuf, cbuf, r):
    def mx_body(i, accs):
        base = i * (L * UMAX)
        return tuple(
            jnp.maximum(a, zbuf[r, pl.ds(base + j * L, L)])
            for j, a in enumerate(accs))
    accs = lax.fori_loop(
        0, VPR // UMAX, mx_body,
        (jnp.full((L,), -jnp.inf, jnp.float32),) * UMAX)
    accm = functools.reduce(jnp.maximum, accs)
    mxs = _splat_max(accm)
    lo0 = mxs - 1.0

    def cp_body(i, cnt_vec):
        base = i * (L * UCP)
        vs, ms, csums, pcs = [], [], [], []
        for j in range(UCP):
            v = zbuf[r, pl.ds(base + j * L, L)]
            m = v > lo0
            vs.append(v)
            ms.append(m)
            csums.append(plsc.cumsum(jnp.where(m, 1, 0)))
            pcs.append(plsc.all_reduce_population_count(m))
        off = cnt_vec
        for j in range(UCP):
            plsc.store_scatter(cbuf, (off + csums[j] - 1,), vs[j], mask=ms[j])
            off = off + pcs[j]
        return off
    cnt_vec = lax.fori_loop(
        0, VPR // UCP, cp_body, jnp.zeros((L,), jnp.int32))
    cnt = cnt_vec[0]
    cbuf[pl.ds(cnt, L)] = lo0 - 1.0
    nv = lax.shift_right_logical(cnt + (L - 1), 4)

    def bis_body(_, lohi):
        lo, hi = lohi
        t = 0.5 * (lo + hi)
        def f_body(i, acc):
            v = cbuf[pl.ds(i * L, L)]
            return acc + jnp.maximum(v - t, 0.0)
        acc = lax.fori_loop(0, nv, f_body, jnp.zeros((L,), jnp.float32))
        ge = _vsum_nonneg(acc) >= 1.0
        return jnp.where(ge, t, lo), jnp.where(ge, hi, t)
    lo, _hi = lax.fori_loop(0, BISECT, bis_body, (lo0, mxs))

    def nf_body(i, acc):
        v = cbuf[pl.ds(i * L, L)]
        m = v > lo
        sacc, kacc = acc
        return (sacc + jnp.where(m, v - lo0, 0.0),
                kacc + jnp.where(m, 1.0, 0.0))
    sacc, kacc = lax.fori_loop(
        0, nv, nf_body,
        (jnp.zeros((L,), jnp.float32), jnp.zeros((L,), jnp.float32)))
    ks = jnp.maximum(_vsum_nonneg(kacc), 1.0)
    tau = (_vsum_nonneg(sacc) + ks * lo0 - 1.0) * _vrecip(ks)

    def out_body(i, carry):
        base = i * (L * UOUT)
        for j in range(UOUT):
            v = zbuf[r, pl.ds(base + j * L, L)]
            obuf[r, pl.ds(base + j * L, L)] = jnp.maximum(v - tau, 0.0)
        return carry
    lax.fori_loop(0, VPR // UOUT, out_body, 0)


def _sc_body(x_hbm, out_hbm, zb0, zb1, ob0, ob1, cbuf, si0, si1, so0, so1):
    wid = lax.axis_index("s") * NC + lax.axis_index("c")
    row0 = wid * ROWS_PER_W
    zb, ob, si, so = (zb0, zb1), (ob0, ob1), (si0, si1), (so0, so1)

    for b in range(2):
        pltpu.async_copy(x_hbm.at[pl.ds(row0 + b * RB, RB)], zb[b], si[b])

    def outer(o, carry):
        for b in range(2):
            t = 2 * o + b
            base = row0 + t * RB
            pltpu.make_async_copy(x_hbm.at[pl.ds(base, RB)], zb[b],
                                  si[b]).wait()

            @pl.when(o > 0)
            def _():
                pltpu.make_async_copy(
                    ob[b], out_hbm.at[pl.ds(base - 2 * RB, RB)],
                    so[b]).wait()

            def row_body(r, c):
                _row_sparsemax(zb[b], ob[b], cbuf, r)
                return c
            lax.fori_loop(0, RB, row_body, 0)
            pltpu.async_copy(ob[b], out_hbm.at[pl.ds(base, RB)], so[b])

            @pl.when(t + 2 < NT)
            def _():
                pltpu.async_copy(x_hbm.at[pl.ds(base + 2 * RB, RB)], zb[b],
                                 si[b])
        return carry
    lax.fori_loop(0, NT // 2, outer, 0)

    for b in range(2):
        base = row0 + (NT - 2 + b) * RB
        pltpu.make_async_copy(ob[b], out_hbm.at[pl.ds(base, RB)],
                              so[b]).wait()


def _tc_body(x_ref, o_ref):
    z = x_ref[...]
    mx = jnp.max(z, axis=-1, keepdims=True)
    lo = mx - 1.0
    hi = mx
    for _ in range(BISECT):
        t = 0.5 * (lo + hi)
        f = jnp.sum(jnp.maximum(z - t, 0.0), axis=-1, keepdims=True)
        ge = f >= 1.0
        lo = jnp.where(ge, t, lo)
        hi = jnp.where(ge, hi, t)
    m = z > lo
    k = jnp.sum(m.astype(jnp.float32), axis=-1, keepdims=True)
    s = jnp.sum(jnp.where(m, z, 0.0), axis=-1, keepdims=True)
    tau = (s - 1.0) / jnp.maximum(k, 1.0)
    o_ref[...] = jnp.maximum(z - tau, 0.0)


def _tc_sparsemax(x):
    n = x.shape[0]
    return pl.pallas_call(
        _tc_body,
        grid=(n // TCB,),
        in_specs=[pl.BlockSpec((TCB, D), lambda i: (i, 0))],
        out_specs=pl.BlockSpec((TCB, D), lambda i: (i, 0)),
        out_shape=jax.ShapeDtypeStruct((n, D), jnp.float32),
    )(x)


@jax.jit
def kernel(input):
    x = input.reshape(NROWS, D)
    mesh = plsc.VectorSubcoreMesh(
        core_axis_name="c", subcore_axis_name="s", num_cores=NC,
        num_subcores=NS)
    run = functools.partial(
        pl.kernel,
        out_type=jax.ShapeDtypeStruct((SC_ROWS, D), jnp.float32),
        mesh=mesh,
        compiler_params=pltpu.CompilerParams(needs_layout_passes=False),
        scratch_types=[
            pltpu.VMEM((RB, D), jnp.float32),
            pltpu.VMEM((RB, D), jnp.float32),
            pltpu.VMEM((RB, D), jnp.float32),
            pltpu.VMEM((RB, D), jnp.float32),
            pltpu.VMEM((D + L,), jnp.float32),
            pltpu.SemaphoreType.DMA,
            pltpu.SemaphoreType.DMA,
            pltpu.SemaphoreType.DMA,
            pltpu.SemaphoreType.DMA,
        ],
    )(_sc_body)
    sc_out = run(x[:SC_ROWS])
    tc_out = _tc_sparsemax(x[SC_ROWS:])
    return jnp.concatenate([sc_out, tc_out], axis=0).reshape(input.shape)


# speedup vs baseline: 3.7956x; 1.2043x over previous
import functools

import jax
import jax.numpy as jnp
from jax import lax
from jax.experimental import pallas as pl
from jax.experimental.pallas import tpu as pltpu
from jax.experimental.pallas import tpu_sc as plsc

L = 16
D = 4096
VPR = D // L
NROWS = 32 * 16 * 16
NC, NS = 2, 16
NW = NC * NS
SC_ROWS = 256
TC_ROWS = NROWS - SC_ROWS
ROWS_PER_W = SC_ROWS // NW
RB = 4
NT = ROWS_PER_W // RB
TCB = 128
BISECT = 12

UMAX = 8
UCP = 8
UOUT = 8


def _splat_max(x):
    return plsc.cummax(lax.rev(plsc.cummax(x), (0,)))


def _vsum_nonneg(x):
    return plsc.cummax(lax.rev(plsc.cumsum(x), (0,)))


def _vrecip(x):
    xi = plsc.bitcast(x, jnp.int32)
    r = plsc.bitcast(jnp.full((L,), 0x7EF311C3, jnp.int32) - xi, jnp.float32)
    for _ in range(3):
        r = r * (2.0 - x * r)
    return r


def _row_sparsemax(zbuf, obuf, cbuf, r):
    def mx_body(i, accs):
        base = i * (L * UMAX)
        return tuple(
            jnp.maximum(a, zbuf[r, pl.ds(base + j * L, L)])
            for j, a in enumerate(accs))
    accs = lax.fori_loop(
        0, VPR // UMAX, mx_body,
        (jnp.full((L,), -jnp.inf, jnp.float32),) * UMAX)
    accm = functools.reduce(jnp.maximum, accs)
    mxs = _splat_max(accm)
    lo0 = mxs - 1.0

    def cp_body(i, cnt_vec):
        base = i * (L * UCP)
        vs, ms, csums, pcs = [], [], [], []
        for j in range(UCP):
            v = zbuf[r, pl.ds(base + j * L, L)]
            m = v > lo0
            vs.append(v)
            ms.append(m)
            csums.append(plsc.cumsum(jnp.where(m, 1, 0)))
            pcs.append(plsc.all_reduce_population_count(m))
        off = cnt_vec
        for j in range(UCP):
            plsc.store_scatter(cbuf, (off + csums[j] - 1,), vs[j], mask=ms[j])
            off = off + pcs[j]
        return off
    cnt_vec = lax.fori_loop(
        0, VPR // UCP, cp_body, jnp.zeros((L,), jnp.int32))
    cnt = cnt_vec[0]
    cbuf[pl.ds(cnt, L)] = lo0 - 1.0
    nv = lax.shift_right_logical(cnt + (L - 1), 4)

    def bis_body(_, lohi):
        lo, hi = lohi
        t = 0.5 * (lo + hi)
        def f_body(i, acc):
            v = cbuf[pl.ds(i * L, L)]
            return acc + jnp.maximum(v - t, 0.0)
        acc = lax.fori_loop(0, nv, f_body, jnp.zeros((L,), jnp.float32))
        ge = _vsum_nonneg(acc) >= 1.0
        return jnp.where(ge, t, lo), jnp.where(ge, hi, t)
    lo, _hi = lax.fori_loop(0, BISECT, bis_body, (lo0, mxs))

    def nf_body(i, acc):
        v = cbuf[pl.ds(i * L, L)]
        m = v > lo
        sacc, kacc = acc
        return (sacc + jnp.where(m, v - lo0, 0.0),
                kacc + jnp.where(m, 1.0, 0.0))
    sacc, kacc = lax.fori_loop(
        0, nv, nf_body,
        (jnp.zeros((L,), jnp.float32), jnp.zeros((L,), jnp.float32)))
    ks = jnp.maximum(_vsum_nonneg(kacc), 1.0)
    tau = (_vsum_nonneg(sacc) + ks * lo0 - 1.0) * _vrecip(ks)

    def out_body(i, carry):
        base = i * (L * UOUT)
        for j in range(UOUT):
            v = zbuf[r, pl.ds(base + j * L, L)]
            obuf[r, pl.ds(base + j * L, L)] = jnp.maximum(v - tau, 0.0)
        return carry
    lax.fori_loop(0, VPR // UOUT, out_body, 0)


def _sc_body(x_hbm, out_hbm, zb0, zb1, ob0, ob1, cbuf, si0, si1, so0, so1):
    wid = lax.axis_index("s") * NC + lax.axis_index("c")
    row0 = wid * ROWS_PER_W
    zb, ob, si, so = (zb0, zb1), (ob0, ob1), (si0, si1), (so0, so1)

    for b in range(2):
        pltpu.async_copy(x_hbm.at[pl.ds(row0 + b * RB, RB)], zb[b], si[b])

    def outer(o, carry):
        for b in range(2):
            t = 2 * o + b
            base = row0 + t * RB
            pltpu.make_async_copy(x_hbm.at[pl.ds(base, RB)], zb[b],
                                  si[b]).wait()

            @pl.when(o > 0)
            def _():
                pltpu.make_async_copy(
                    ob[b], out_hbm.at[pl.ds(base - 2 * RB, RB)],
                    so[b]).wait()

            def row_body(r, c):
                _row_sparsemax(zb[b], ob[b], cbuf, r)
                return c
            lax.fori_loop(0, RB, row_body, 0)
            pltpu.async_copy(ob[b], out_hbm.at[pl.ds(base, RB)], so[b])

            @pl.when(t + 2 < NT)
            def _():
                pltpu.async_copy(x_hbm.at[pl.ds(base + 2 * RB, RB)], zb[b],
                                 si[b])
        return carry
    lax.fori_loop(0, NT // 2, outer, 0)

    for b in range(2):
        base = row0 + (NT - 2 + b) * RB
        pltpu.make_async_copy(ob[b], out_hbm.at[pl.ds(base, RB)],
                              so[b]).wait()


def _tc_body(x_ref, o_ref):
    z = x_ref[...]
    mx = jnp.max(z, axis=-1, keepdims=True)
    lo = mx - 1.0
    hi = mx
    for _ in range(BISECT):
        t = 0.5 * (lo + hi)
        f = jnp.sum(jnp.maximum(z - t, 0.0), axis=-1, keepdims=True)
        ge = f >= 1.0
        lo = jnp.where(ge, t, lo)
        hi = jnp.where(ge, hi, t)
    m = z > lo
    k = jnp.sum(m.astype(jnp.float32), axis=-1, keepdims=True)
    s = jnp.sum(jnp.where(m, z, 0.0), axis=-1, keepdims=True)
    tau = (s - 1.0) / jnp.maximum(k, 1.0)
    o_ref[...] = jnp.maximum(z - tau, 0.0)


def _tc_sparsemax(x):
    n = x.shape[0]
    return pl.pallas_call(
        _tc_body,
        grid=(n // TCB,),
        in_specs=[pl.BlockSpec((TCB, D), lambda i: (i, 0))],
        out_specs=pl.BlockSpec((TCB, D), lambda i: (i, 0)),
        out_shape=jax.ShapeDtypeStruct((n, D), jnp.float32),
    )(x)


@jax.jit
def kernel(input):
    x = input.reshape(NROWS, D)
    mesh = plsc.VectorSubcoreMesh(
        core_axis_name="c", subcore_axis_name="s", num_cores=NC,
        num_subcores=NS)
    run = functools.partial(
        pl.kernel,
        out_type=jax.ShapeDtypeStruct((SC_ROWS, D), jnp.float32),
        mesh=mesh,
        compiler_params=pltpu.CompilerParams(needs_layout_passes=False),
        scratch_types=[
            pltpu.VMEM((RB, D), jnp.float32),
            pltpu.VMEM((RB, D), jnp.float32),
            pltpu.VMEM((RB, D), jnp.float32),
            pltpu.VMEM((RB, D), jnp.float32),
            pltpu.VMEM((D + L,), jnp.float32),
            pltpu.SemaphoreType.DMA,
            pltpu.SemaphoreType.DMA,
            pltpu.SemaphoreType.DMA,
            pltpu.SemaphoreType.DMA,
        ],
    )(_sc_body)
    sc_out = run(x[:SC_ROWS])
    tc_out = _tc_sparsemax(x[SC_ROWS:])
    return jnp.concatenate([sc_out, tc_out], axis=0).reshape(input.shape)


# speedup vs baseline: 4.2901x; 1.1303x over previous
import functools

import jax
import jax.numpy as jnp
from jax import lax
from jax.experimental import pallas as pl
from jax.experimental.pallas import tpu as pltpu
from jax.experimental.pallas import tpu_sc as plsc

L = 16
D = 4096
VPR = D // L
NROWS = 32 * 16 * 16
NC, NS = 2, 16
NW = NC * NS
SC_ROWS = 2304
TC_ROWS = NROWS - SC_ROWS
ROWS_PER_W = SC_ROWS // NW
RB = 4
NT = ROWS_PER_W // RB
TCB = 128
BISECT = 12

UMAX = 8
UCP = 8
UOUT = 8


def _splat_max(x):
    return plsc.cummax(lax.rev(plsc.cummax(x), (0,)))


def _vsum_nonneg(x):
    return plsc.cummax(lax.rev(plsc.cumsum(x), (0,)))


def _vrecip(x):
    xi = plsc.bitcast(x, jnp.int32)
    r = plsc.bitcast(jnp.full((L,), 0x7EF311C3, jnp.int32) - xi, jnp.float32)
    for _ in range(3):
        r = r * (2.0 - x * r)
    return r


def _row_sparsemax(zbuf, obuf, cbuf, r):
    def mx_body(i, accs):
        base = i * (L * UMAX)
        return tuple(
            jnp.maximum(a, zbuf[r, pl.ds(base + j * L, L)])
            for j, a in enumerate(accs))
    accs = lax.fori_loop(
        0, VPR // UMAX, mx_body,
        (jnp.full((L,), -jnp.inf, jnp.float32),) * UMAX)
    accm = functools.reduce(jnp.maximum, accs)
    mxs = _splat_max(accm)
    lo0 = mxs - 1.0

    def cp_body(i, cnt_vec):
        base = i * (L * UCP)
        vs, ms, csums, pcs = [], [], [], []
        for j in range(UCP):
            v = zbuf[r, pl.ds(base + j * L, L)]
            m = v > lo0
            vs.append(v)
            ms.append(m)
            csums.append(plsc.cumsum(jnp.where(m, 1, 0)))
            pcs.append(plsc.all_reduce_population_count(m))
        off = cnt_vec
        for j in range(UCP):
            plsc.store_scatter(cbuf, (off + csums[j] - 1,), vs[j], mask=ms[j])
            off = off + pcs[j]
        return off
    cnt_vec = lax.fori_loop(
        0, VPR // UCP, cp_body, jnp.zeros((L,), jnp.int32))
    cnt = cnt_vec[0]
    cbuf[pl.ds(cnt, L)] = lo0 - 1.0
    nv = lax.shift_right_logical(cnt + (L - 1), 4)

    def bis_body(_, lohi):
        lo, hi = lohi
        t = 0.5 * (lo + hi)
        def f_body(i, acc):
            v = cbuf[pl.ds(i * L, L)]
            return acc + jnp.maximum(v - t, 0.0)
        acc = lax.fori_loop(0, nv, f_body, jnp.zeros((L,), jnp.float32))
        ge = _vsum_nonneg(acc) >= 1.0
        return jnp.where(ge, t, lo), jnp.where(ge, hi, t)
    lo, _hi = lax.fori_loop(0, BISECT, bis_body, (lo0, mxs))

    def nf_body(i, acc):
        v = cbuf[pl.ds(i * L, L)]
        m = v > lo
        sacc, kacc = acc
        return (sacc + jnp.where(m, v - lo0, 0.0),
                kacc + jnp.where(m, 1.0, 0.0))
    sacc, kacc = lax.fori_loop(
        0, nv, nf_body,
        (jnp.zeros((L,), jnp.float32), jnp.zeros((L,), jnp.float32)))
    ks = jnp.maximum(_vsum_nonneg(kacc), 1.0)
    tau = (_vsum_nonneg(sacc) + ks * lo0 - 1.0) * _vrecip(ks)

    def out_body(i, carry):
        base = i * (L * UOUT)
        for j in range(UOUT):
            v = zbuf[r, pl.ds(base + j * L, L)]
            obuf[r, pl.ds(base + j * L, L)] = jnp.maximum(v - tau, 0.0)
        return carry
    lax.fori_loop(0, VPR // UOUT, out_body, 0)


def _sc_body(x_hbm, out_hbm, zb0, zb1, ob0, ob1, cbuf, si0, si1, so0, so1):
    wid = lax.axis_index("s") * NC + lax.axis_index("c")
    row0 = wid * ROWS_PER_W
    zb, ob, si, so = (zb0, zb1), (ob0, ob1), (si0, si1), (so0, so1)

    for b in range(2):
        pltpu.async_copy(x_hbm.at[pl.ds(row0 + b * RB, RB)], zb[b], si[b])

    def outer(o, carry):
        for b in range(2):
            t = 2 * o + b
            base = row0 + t * RB
            pltpu.make_async_copy(x_hbm.at[pl.ds(base, RB)], zb[b],
                                  si[b]).wait()

            @pl.when(o > 0)
            def _():
                pltpu.make_async_copy(
                    ob[b], out_hbm.at[pl.ds(base - 2 * RB, RB)],
                    so[b]).wait()

            def row_body(r, c):
                _row_sparsemax(zb[b], ob[b], cbuf, r)
                return c
            lax.fori_loop(0, RB, row_body, 0)
            pltpu.async_copy(ob[b], out_hbm.at[pl.ds(base, RB)], so[b])

            @pl.when(t + 2 < NT)
            def _():
                pltpu.async_copy(x_hbm.at[pl.ds(base + 2 * RB, RB)], zb[b],
                                 si[b])
        return carry
    lax.fori_loop(0, NT // 2, outer, 0)

    for b in range(2):
        base = row0 + (NT - 2 + b) * RB
        pltpu.make_async_copy(ob[b], out_hbm.at[pl.ds(base, RB)],
                              so[b]).wait()


def _tc_body(x_ref, o_ref):
    z = x_ref[...]
    mx = jnp.max(z, axis=-1, keepdims=True)
    lo = mx - 1.0
    hi = mx
    for _ in range(BISECT):
        t = 0.5 * (lo + hi)
        f = jnp.sum(jnp.maximum(z - t, 0.0), axis=-1, keepdims=True)
        ge = f >= 1.0
        lo = jnp.where(ge, t, lo)
        hi = jnp.where(ge, hi, t)
    m = z > lo
    k = jnp.sum(m.astype(jnp.float32), axis=-1, keepdims=True)
    s = jnp.sum(jnp.where(m, z, 0.0), axis=-1, keepdims=True)
    tau = (s - 1.0) / jnp.maximum(k, 1.0)
    o_ref[...] = jnp.maximum(z - tau, 0.0)


def _tc_sparsemax(x):
    n = x.shape[0]
    return pl.pallas_call(
        _tc_body,
        grid=(n // TCB,),
        in_specs=[pl.BlockSpec((TCB, D), lambda i: (i, 0))],
        out_specs=pl.BlockSpec((TCB, D), lambda i: (i, 0)),
        out_shape=jax.ShapeDtypeStruct((n, D), jnp.float32),
    )(x)


@jax.jit
def kernel(input):
    x = input.reshape(NROWS, D)
    mesh = plsc.VectorSubcoreMesh(
        core_axis_name="c", subcore_axis_name="s", num_cores=NC,
        num_subcores=NS)
    run = functools.partial(
        pl.kernel,
        out_type=jax.ShapeDtypeStruct((SC_ROWS, D), jnp.float32),
        mesh=mesh,
        compiler_params=pltpu.CompilerParams(needs_layout_passes=False),
        scratch_types=[
            pltpu.VMEM((RB, D), jnp.float32),
            pltpu.VMEM((RB, D), jnp.float32),
            pltpu.VMEM((RB, D), jnp.float32),
            pltpu.VMEM((RB, D), jnp.float32),
            pltpu.VMEM((D + L,), jnp.float32),
            pltpu.SemaphoreType.DMA,
            pltpu.SemaphoreType.DMA,
            pltpu.SemaphoreType.DMA,
            pltpu.SemaphoreType.DMA,
        ],
    )(_sc_body)
    sc_out = run(x[:SC_ROWS])
    tc_out = _tc_sparsemax(x[SC_ROWS:])
    return jnp.concatenate([sc_out, tc_out], axis=0).reshape(input.shape)


# speedup vs baseline: 5.1403x; 1.1982x over previous
import functools

import jax
import jax.numpy as jnp
from jax import lax
from jax.experimental import pallas as pl
from jax.experimental.pallas import tpu as pltpu
from jax.experimental.pallas import tpu_sc as plsc

L = 16
D = 4096
VPR = D // L
NROWS = 32 * 16 * 16
NC, NS = 2, 16
NW = NC * NS
SC_ROWS = 2304
TC_ROWS = NROWS - SC_ROWS
ROWS_PER_W = SC_ROWS // NW
RB = 4
NT = ROWS_PER_W // RB
TCB = 128
BISECT = 12

UMAX = 8
UCP = 8
UOUT = 8


def _splat_max(x):
    return plsc.cummax(lax.rev(plsc.cummax(x), (0,)))


def _vsum_nonneg(x):
    return plsc.cummax(lax.rev(plsc.cumsum(x), (0,)))


def _vrecip(x):
    xi = plsc.bitcast(x, jnp.int32)
    r = plsc.bitcast(jnp.full((L,), 0x7EF311C3, jnp.int32) - xi, jnp.float32)
    for _ in range(3):
        r = r * (2.0 - x * r)
    return r


def _row_sparsemax(zbuf, obuf, cbuf, r):
    def mx_body(i, accs):
        base = i * (L * UMAX)
        return tuple(
            jnp.maximum(a, zbuf[r, pl.ds(base + j * L, L)])
            for j, a in enumerate(accs))
    accs = lax.fori_loop(
        0, VPR // UMAX, mx_body,
        (jnp.full((L,), -jnp.inf, jnp.float32),) * UMAX)
    accm = functools.reduce(jnp.maximum, accs)
    mxs = _splat_max(accm)
    lo0 = mxs - 1.0

    def cp_body(i, cnt_vec):
        base = i * (L * UCP)
        vs, ms, csums, pcs = [], [], [], []
        for j in range(UCP):
            v = zbuf[r, pl.ds(base + j * L, L)]
            m = v > lo0
            vs.append(v)
            ms.append(m)
            csums.append(plsc.cumsum(jnp.where(m, 1, 0)))
            pcs.append(plsc.all_reduce_population_count(m))
        off = cnt_vec
        for j in range(UCP):
            plsc.store_scatter(cbuf, (off + csums[j] - 1,), vs[j], mask=ms[j])
            off = off + pcs[j]
        return off
    cnt_vec = lax.fori_loop(
        0, VPR // UCP, cp_body, jnp.zeros((L,), jnp.int32))
    cnt = cnt_vec[0]
    cbuf[pl.ds(cnt, L)] = lo0 - 1.0
    nv = lax.shift_right_logical(cnt + (L - 1), 4)

    def bis_body(_, lohi):
        lo, hi = lohi
        t = 0.5 * (lo + hi)
        def f_body(i, acc):
            v = cbuf[pl.ds(i * L, L)]
            return acc + jnp.maximum(v - t, 0.0)
        acc = lax.fori_loop(0, nv, f_body, jnp.zeros((L,), jnp.float32))
        ge = _vsum_nonneg(acc) >= 1.0
        return jnp.where(ge, t, lo), jnp.where(ge, hi, t)
    lo, _hi = lax.fori_loop(0, BISECT, bis_body, (lo0, mxs))

    def nf_body(i, acc):
        v = cbuf[pl.ds(i * L, L)]
        m = v > lo
        sacc, kacc = acc
        return (sacc + jnp.where(m, v - lo0, 0.0),
                kacc + jnp.where(m, 1.0, 0.0))
    sacc, kacc = lax.fori_loop(
        0, nv, nf_body,
        (jnp.zeros((L,), jnp.float32), jnp.zeros((L,), jnp.float32)))
    ks = jnp.maximum(_vsum_nonneg(kacc), 1.0)
    tau = (_vsum_nonneg(sacc) + ks * lo0 - 1.0) * _vrecip(ks)

    def out_body(i, carry):
        base = i * (L * UOUT)
        for j in range(UOUT):
            v = zbuf[r, pl.ds(base + j * L, L)]
            obuf[r, pl.ds(base + j * L, L)] = jnp.maximum(v - tau, 0.0)
        return carry
    lax.fori_loop(0, VPR // UOUT, out_body, 0)


def _sc_body(x_hbm, out_hbm, zb0, zb1, ob0, ob1, cbuf, si0, si1, so0, so1):
    wid = lax.axis_index("s") * NC + lax.axis_index("c")
    row0 = wid * ROWS_PER_W
    zb, ob, si, so = (zb0, zb1), (ob0, ob1), (si0, si1), (so0, so1)

    for b in range(2):
        pltpu.async_copy(x_hbm.at[pl.ds(row0 + b * RB, RB)], zb[b], si[b])

    def outer(o, carry):
        for b in range(2):
            t = 2 * o + b
            base = row0 + t * RB
            pltpu.make_async_copy(x_hbm.at[pl.ds(base, RB)], zb[b],
                                  si[b]).wait()

            @pl.when(o > 0)
            def _():
                pltpu.make_async_copy(
                    ob[b], out_hbm.at[pl.ds(base - 2 * RB, RB)],
                    so[b]).wait()

            def row_body(r, c):
                _row_sparsemax(zb[b], ob[b], cbuf, r)
                return c
            lax.fori_loop(0, RB, row_body, 0)
            pltpu.async_copy(ob[b], out_hbm.at[pl.ds(base, RB)], so[b])

            @pl.when(t + 2 < NT)
            def _():
                pltpu.async_copy(x_hbm.at[pl.ds(base + 2 * RB, RB)], zb[b],
                                 si[b])
        return carry
    lax.fori_loop(0, NT // 2, outer, 0)

    for b in range(2):
        base = row0 + (NT - 2 + b) * RB
        pltpu.make_async_copy(ob[b], out_hbm.at[pl.ds(base, RB)],
                              so[b]).wait()


def _tc_body(x_ref, o_ref):
    z = x_ref[...]
    mx = jnp.max(z, axis=-1, keepdims=True)
    lo = mx - 1.0
    hi = mx
    for _ in range(BISECT):
        t = 0.5 * (lo + hi)
        f = jnp.sum(jnp.maximum(z - t, 0.0), axis=-1, keepdims=True)
        ge = f >= 1.0
        lo = jnp.where(ge, t, lo)
        hi = jnp.where(ge, hi, t)
    m = z > lo
    k = jnp.sum(m.astype(jnp.float32), axis=-1, keepdims=True)
    s = jnp.sum(jnp.where(m, z, 0.0), axis=-1, keepdims=True)
    tau = (s - 1.0) / jnp.maximum(k, 1.0)
    o_ref[...] = jnp.maximum(z - tau, 0.0)


def _tc_sparsemax(x):
    return pl.pallas_call(
        _tc_body,
        grid=(TC_ROWS // TCB,),
        in_specs=[pl.BlockSpec((TCB, D), lambda i: (i + SC_ROWS // TCB, 0))],
        out_specs=pl.BlockSpec((TCB, D), lambda i: (i, 0)),
        out_shape=jax.ShapeDtypeStruct((TC_ROWS, D), jnp.float32),
    )(x)


@jax.jit
def kernel(input):
    x = input.reshape(NROWS, D)
    mesh = plsc.VectorSubcoreMesh(
        core_axis_name="c", subcore_axis_name="s", num_cores=NC,
        num_subcores=NS)
    run = functools.partial(
        pl.kernel,
        out_type=jax.ShapeDtypeStruct((SC_ROWS, D), jnp.float32),
        mesh=mesh,
        compiler_params=pltpu.CompilerParams(needs_layout_passes=False),
        scratch_types=[
            pltpu.VMEM((RB, D), jnp.float32),
            pltpu.VMEM((RB, D), jnp.float32),
            pltpu.VMEM((RB, D), jnp.float32),
            pltpu.VMEM((RB, D), jnp.float32),
            pltpu.VMEM((D + L,), jnp.float32),
            pltpu.SemaphoreType.DMA,
            pltpu.SemaphoreType.DMA,
            pltpu.SemaphoreType.DMA,
            pltpu.SemaphoreType.DMA,
        ],
    )(_sc_body)
    sc_out = run(x)
    tc_out = _tc_sparsemax(x)
    return jnp.concatenate([sc_out, tc_out], axis=0).reshape(input.shape)
